# baseline (speedup 1.0000x reference)
import jax
import jax.numpy as jnp
from jax.experimental import pallas as pl


def _mean_body(x_ref, o_ref):
    x = x_ref[...]
    m = jnp.mean(x, axis=1, keepdims=True)
    o_ref[...] = jnp.broadcast_to(m, x.shape)


def kernel(in_features, seq_start_end):
    del seq_start_end
    n, a, d = in_features.shape
    block = 256
    grid = (n // block,)
    return pl.pallas_call(
        _mean_body,
        grid=grid,
        in_specs=[pl.BlockSpec((block, a, d), lambda i: (i, 0, 0))],
        out_specs=pl.BlockSpec((block, a, d), lambda i: (i, 0, 0)),
        out_shape=jax.ShapeDtypeStruct((n, a, d), in_features.dtype),
    )(in_features)
